# Initial kernel scaffold; baseline (speedup 1.0000x reference)
import jax
import jax.numpy as jnp
from jax.experimental import pallas as pl


def kernel(x, edge_index, batch, W1, b1, W2, b2):
    raise NotImplementedError("write your pallas kernel here")


# speedup vs baseline: 19.6249x; 19.6249x over previous
import functools

import jax
import jax.numpy as jnp
from jax import lax
from jax.experimental import pallas as pl
from jax.experimental.pallas import tpu as pltpu
from jax.experimental.pallas import tpu_sc as plsc

N = 10000
D = 128
DH = D // 2
E = 320000

NC = 2
NS = 16
NW = NC * NS
EPT = E // NW
K = 125
NCH = EPT // K
NPAIR = NCH // 2
NP = 10240
RPT = NP // NS
DW = 16

_MESH = dict(core_axis_name="c", subcore_axis_name="s", num_cores=NC,
             num_subcores=NS)


@functools.partial(
    pl.kernel,
    out_type=jax.ShapeDtypeStruct((NC, NP, DW), jnp.float32),
    mesh=plsc.VectorSubcoreMesh(**_MESH),
    compiler_params=pltpu.CompilerParams(use_tc_tiling_on_sc=False),
    scratch_types=[
        pltpu.VMEM((NCH, K), jnp.int32),
        pltpu.VMEM((K, DW), jnp.float32),
        pltpu.VMEM_SHARED((NP, DW), jnp.float32),
    ],
)
def _deg_kernel(dst_hbm, ones_hbm, zeros_hbm, out_hbm, idx_d, ones_v, acc):
    c = lax.axis_index("c")
    s = lax.axis_index("s")
    wid = c * NS + s

    pltpu.sync_copy(dst_hbm.at[wid], idx_d)
    pltpu.sync_copy(ones_hbm, ones_v)
    pltpu.sync_copy(zeros_hbm.at[pl.ds(s * RPT, RPT)],
                    acc.at[pl.ds(s * RPT, RPT)])
    plsc.subcore_barrier()

    def body(j, _):
        pltpu.sync_copy(ones_v, acc.at[idx_d.at[j]], add=True)
        return _

    lax.fori_loop(0, NCH, body, None)
    plsc.subcore_barrier()

    pltpu.sync_copy(acc.at[pl.ds(s * RPT, RPT)],
                    out_hbm.at[c, pl.ds(s * RPT, RPT)])


@functools.partial(
    pl.kernel,
    out_type=jax.ShapeDtypeStruct((NC, 2, NP, DH), jnp.float32),
    mesh=plsc.VectorSubcoreMesh(**_MESH),
    compiler_params=pltpu.CompilerParams(use_tc_tiling_on_sc=False),
    scratch_types=[
        pltpu.VMEM((NCH, K), jnp.int32),
        pltpu.VMEM((NCH, K), jnp.int32),
        pltpu.VMEM((K, DH), jnp.float32),
        pltpu.VMEM((K, DH), jnp.float32),
        pltpu.SemaphoreType.DMA,
        pltpu.SemaphoreType.DMA,
        pltpu.VMEM_SHARED((NP, DH), jnp.float32),
    ],
)
def _edge_kernel(g_hbm, src_hbm, dst_hbm, zeros_hbm, out_hbm,
                 idx_s, idx_d, rows_a, rows_b, sem_a, sem_b, acc):
    c = lax.axis_index("c")
    s = lax.axis_index("s")
    wid = c * NS + s

    pltpu.sync_copy(src_hbm.at[wid], idx_s)
    pltpu.sync_copy(dst_hbm.at[wid], idx_d)

    for h in range(2):
        pltpu.sync_copy(zeros_hbm.at[pl.ds(s * RPT, RPT)],
                        acc.at[pl.ds(s * RPT, RPT)])
        plsc.subcore_barrier()

        gh = g_hbm.at[h]
        pltpu.async_copy(gh.at[idx_s.at[0]], rows_a, sem_a)

        def body(j, _):
            e = 2 * j
            o = e + 1
            pltpu.make_async_copy(gh.at[idx_s.at[e]], rows_a, sem_a).wait()
            db = pltpu.async_copy(gh.at[idx_s.at[o]], rows_b, sem_b)
            pltpu.sync_copy(rows_a, acc.at[idx_d.at[e]], add=True)
            db.wait()

            @pl.when(j < NPAIR - 1)
            def _():
                pltpu.async_copy(gh.at[idx_s.at[e + 2]], rows_a, sem_a)

            pltpu.sync_copy(rows_b, acc.at[idx_d.at[o]], add=True)
            return _

        lax.fori_loop(0, NPAIR, body, None)
        plsc.subcore_barrier()

        pltpu.sync_copy(acc.at[pl.ds(s * RPT, RPT)],
                        out_hbm.at[c, h, pl.ds(s * RPT, RPT)])


NB = 10
BR = N // NB


def _tc1_body(degp_ref, x_ref, w1_ref, dis_ref, g1_ref):
    deg = degp_ref[0, :, :1] + degp_ref[1, :, :1] + 1.0
    dis = lax.rsqrt(deg)
    h1 = jnp.dot(x_ref[...], w1_ref[...],
                 preferred_element_type=jnp.float32,
                 precision=lax.Precision.HIGHEST)
    g1 = dis * h1
    dis_ref[...] = dis
    g1_ref[0] = g1[:, :DH]
    g1_ref[1] = g1[:, DH:]


def _combine(sp_ref, g_ref):
    s_lo = sp_ref[0, 0] + sp_ref[1, 0] + g_ref[0]
    s_hi = sp_ref[0, 1] + sp_ref[1, 1] + g_ref[1]
    return jnp.concatenate([s_lo, s_hi], axis=1)


def _tc2_body(s1p_ref, g1_ref, dis_ref, b1_ref, w2_ref, g2_ref):
    dis = dis_ref[...]
    t = _combine(s1p_ref, g1_ref) * dis + b1_ref[...]
    z = jnp.maximum(t, 0.0)
    h2 = jnp.dot(z, w2_ref[...],
                 preferred_element_type=jnp.float32,
                 precision=lax.Precision.HIGHEST)
    g2 = dis * h2
    g2_ref[0] = g2[:, :DH]
    g2_ref[1] = g2[:, DH:]


def _tc3_body(s2p_ref, g2_ref, dis_ref, b2_ref, out_ref):
    out_ref[...] = _combine(s2p_ref, g2_ref) * dis_ref[...] + b2_ref[...]


_spec_degp = pl.BlockSpec((NC, BR, DW), lambda i: (0, i, 0))
_spec_sp = pl.BlockSpec((NC, 2, BR, DH), lambda i: (0, 0, i, 0))
_spec_g = pl.BlockSpec((2, BR, DH), lambda i: (0, i, 0))
_spec_dis = pl.BlockSpec((BR, 1), lambda i: (i, 0))
_spec_row = pl.BlockSpec((BR, D), lambda i: (i, 0))
_spec_b = pl.BlockSpec((1, D), lambda i: (0, 0))
_spec_w = pl.BlockSpec((D, D), lambda i: (0, 0))

_tc1 = pl.pallas_call(
    _tc1_body,
    grid=(NB,),
    in_specs=[_spec_degp, _spec_row, _spec_w],
    out_specs=(_spec_dis, _spec_g),
    out_shape=(jax.ShapeDtypeStruct((N, 1), jnp.float32),
               jax.ShapeDtypeStruct((2, N, DH), jnp.float32)),
)

_tc2 = pl.pallas_call(
    _tc2_body,
    grid=(NB,),
    in_specs=[_spec_sp, _spec_g, _spec_dis, _spec_b, _spec_w],
    out_specs=_spec_g,
    out_shape=jax.ShapeDtypeStruct((2, N, DH), jnp.float32),
)

_tc3 = pl.pallas_call(
    _tc3_body,
    grid=(NB,),
    in_specs=[_spec_sp, _spec_g, _spec_dis, _spec_b],
    out_specs=_spec_row,
    out_shape=jax.ShapeDtypeStruct((N, D), jnp.float32),
)


def kernel(x, edge_index, batch, W1, b1, W2, b2):
    del batch
    src = edge_index[0].reshape(NW, NCH, K)
    dst = edge_index[1].reshape(NW, NCH, K)
    zeros_feat = jnp.zeros((NP, DH), jnp.float32)
    zeros_deg = jnp.zeros((NP, DW), jnp.float32)
    ones_chunk = jnp.ones((K, DW), jnp.float32)

    degp = _deg_kernel(dst, ones_chunk, zeros_deg)
    dis, g1 = _tc1(degp, x, W1)
    s1p = _edge_kernel(g1, src, dst, zeros_feat)
    g2 = _tc2(s1p, g1, dis, b1.reshape(1, D), W2)
    s2p = _edge_kernel(g2, src, dst, zeros_feat)
    out = _tc3(s2p, g2, dis, b2.reshape(1, D))
    return out


# speedup vs baseline: 27.7733x; 1.4152x over previous
import functools

import jax
import jax.numpy as jnp
from jax import lax
from jax.experimental import pallas as pl
from jax.experimental.pallas import tpu as pltpu
from jax.experimental.pallas import tpu_sc as plsc

N = 10000
D = 128
DH = D // 2
E = 320000

NC = 2
NS = 16
NW = NC * NS
EPT = E // NW
K = 125
NCH = EPT // K
NBUF = 8
LOOK = 6
LAG = NBUF - LOOK
NP = 10240
RPT = NP // NS
DW = 16

_MESH = dict(core_axis_name="c", subcore_axis_name="s", num_cores=NC,
             num_subcores=NS)


@functools.partial(
    pl.kernel,
    out_type=jax.ShapeDtypeStruct((NC, NP, DW), jnp.float32),
    mesh=plsc.VectorSubcoreMesh(**_MESH),
    compiler_params=pltpu.CompilerParams(use_tc_tiling_on_sc=False),
    scratch_types=[
        pltpu.VMEM((NCH, K), jnp.int32),
        pltpu.VMEM((K, DW), jnp.float32),
        pltpu.VMEM_SHARED((NP, DW), jnp.float32),
    ],
)
def _deg_kernel(dst_hbm, ones_hbm, zeros_hbm, out_hbm, idx_d, ones_v, acc):
    c = lax.axis_index("c")
    s = lax.axis_index("s")
    wid = c * NS + s

    pltpu.sync_copy(dst_hbm.at[wid], idx_d)
    pltpu.sync_copy(ones_hbm, ones_v)
    pltpu.sync_copy(zeros_hbm.at[pl.ds(s * RPT, RPT)],
                    acc.at[pl.ds(s * RPT, RPT)])
    plsc.subcore_barrier()

    def body(j, _):
        pltpu.sync_copy(ones_v, acc.at[idx_d.at[j]], add=True)
        return _

    lax.fori_loop(0, NCH, body, None)
    plsc.subcore_barrier()

    pltpu.sync_copy(acc.at[pl.ds(s * RPT, RPT)],
                    out_hbm.at[c, pl.ds(s * RPT, RPT)])


@functools.partial(
    pl.kernel,
    out_type=jax.ShapeDtypeStruct((NC, 2, NP, DH), jnp.float32),
    mesh=plsc.VectorSubcoreMesh(**_MESH),
    compiler_params=pltpu.CompilerParams(use_tc_tiling_on_sc=False),
    scratch_types=[
        pltpu.VMEM((NCH, K), jnp.int32),
        pltpu.VMEM((NCH, K), jnp.int32),
        [pltpu.VMEM((K, DH), jnp.float32) for _ in range(NBUF)],
        [pltpu.SemaphoreType.DMA for _ in range(NBUF)],
        [pltpu.SemaphoreType.DMA for _ in range(NBUF)],
        pltpu.VMEM_SHARED((NP, DH), jnp.float32),
    ],
)
def _edge_kernel(g_hbm, src_hbm, dst_hbm, zeros_hbm, out_hbm,
                 idx_s, idx_d, rows, gsem, ssem, acc):
    c = lax.axis_index("c")
    s = lax.axis_index("s")
    wid = c * NS + s

    pltpu.sync_copy(src_hbm.at[wid], idx_s)
    pltpu.sync_copy(dst_hbm.at[wid], idx_d)

    for h in range(2):
        pltpu.sync_copy(zeros_hbm.at[pl.ds(s * RPT, RPT)],
                        acc.at[pl.ds(s * RPT, RPT)])
        plsc.subcore_barrier()

        gh = g_hbm.at[h]
        for b in range(LOOK):
            pltpu.async_copy(gh.at[idx_s.at[b]], rows[b], gsem[b])

        def body(j, _):
            for bi in range(NBUF):
                ch = NBUF * j + bi
                pltpu.make_async_copy(gh.at[idx_s.at[ch]],
                                      rows[bi], gsem[bi]).wait()
                pltpu.async_copy(rows[bi], acc.at[idx_d.at[ch]], ssem[bi],
                                 add=True)
                bw = (bi - LAG) % NBUF

                @pl.when(ch >= LAG)
                def _():
                    pltpu.make_async_copy(rows[bw], acc.at[idx_d.at[ch]],
                                          ssem[bw]).wait()

                bg = (bi + LOOK) % NBUF

                @pl.when(ch + LOOK < NCH)
                def _():
                    pltpu.async_copy(gh.at[idx_s.at[ch + LOOK]],
                                     rows[bg], gsem[bg])
            return _

        lax.fori_loop(0, NCH // NBUF, body, None)
        for ch in range(NCH - LAG, NCH):
            bi = ch % NBUF
            pltpu.make_async_copy(rows[bi], acc.at[idx_d.at[NCH - 1]],
                                  ssem[bi]).wait()
        plsc.subcore_barrier()

        pltpu.sync_copy(acc.at[pl.ds(s * RPT, RPT)],
                        out_hbm.at[c, h, pl.ds(s * RPT, RPT)])


NB = 10
BR = N // NB


def _tc1_body(degp_ref, x_ref, w1_ref, dis_ref, g1_ref):
    deg = degp_ref[0, :, :1] + degp_ref[1, :, :1] + 1.0
    dis = lax.rsqrt(deg)
    h1 = jnp.dot(x_ref[...], w1_ref[...],
                 preferred_element_type=jnp.float32,
                 precision=lax.Precision.HIGHEST)
    g1 = dis * h1
    dis_ref[...] = dis
    g1_ref[0] = g1[:, :DH]
    g1_ref[1] = g1[:, DH:]


def _combine(sp_ref, g_ref):
    s_lo = sp_ref[0, 0] + sp_ref[1, 0] + g_ref[0]
    s_hi = sp_ref[0, 1] + sp_ref[1, 1] + g_ref[1]
    return jnp.concatenate([s_lo, s_hi], axis=1)


def _tc2_body(s1p_ref, g1_ref, dis_ref, b1_ref, w2_ref, g2_ref):
    dis = dis_ref[...]
    t = _combine(s1p_ref, g1_ref) * dis + b1_ref[...]
    z = jnp.maximum(t, 0.0)
    h2 = jnp.dot(z, w2_ref[...],
                 preferred_element_type=jnp.float32,
                 precision=lax.Precision.HIGHEST)
    g2 = dis * h2
    g2_ref[0] = g2[:, :DH]
    g2_ref[1] = g2[:, DH:]


def _tc3_body(s2p_ref, g2_ref, dis_ref, b2_ref, out_ref):
    out_ref[...] = _combine(s2p_ref, g2_ref) * dis_ref[...] + b2_ref[...]


_spec_degp = pl.BlockSpec((NC, BR, DW), lambda i: (0, i, 0))
_spec_sp = pl.BlockSpec((NC, 2, BR, DH), lambda i: (0, 0, i, 0))
_spec_g = pl.BlockSpec((2, BR, DH), lambda i: (0, i, 0))
_spec_dis = pl.BlockSpec((BR, 1), lambda i: (i, 0))
_spec_row = pl.BlockSpec((BR, D), lambda i: (i, 0))
_spec_b = pl.BlockSpec((1, D), lambda i: (0, 0))
_spec_w = pl.BlockSpec((D, D), lambda i: (0, 0))

_tc1 = pl.pallas_call(
    _tc1_body,
    grid=(NB,),
    in_specs=[_spec_degp, _spec_row, _spec_w],
    out_specs=(_spec_dis, _spec_g),
    out_shape=(jax.ShapeDtypeStruct((N, 1), jnp.float32),
               jax.ShapeDtypeStruct((2, N, DH), jnp.float32)),
)

_tc2 = pl.pallas_call(
    _tc2_body,
    grid=(NB,),
    in_specs=[_spec_sp, _spec_g, _spec_dis, _spec_b, _spec_w],
    out_specs=_spec_g,
    out_shape=jax.ShapeDtypeStruct((2, N, DH), jnp.float32),
)

_tc3 = pl.pallas_call(
    _tc3_body,
    grid=(NB,),
    in_specs=[_spec_sp, _spec_g, _spec_dis, _spec_b],
    out_specs=_spec_row,
    out_shape=jax.ShapeDtypeStruct((N, D), jnp.float32),
)


def kernel(x, edge_index, batch, W1, b1, W2, b2):
    del batch
    src = edge_index[0].reshape(NW, NCH, K)
    dst = edge_index[1].reshape(NW, NCH, K)
    zeros_feat = jnp.zeros((NP, DH), jnp.float32)
    zeros_deg = jnp.zeros((NP, DW), jnp.float32)
    ones_chunk = jnp.ones((K, DW), jnp.float32)

    degp = _deg_kernel(dst, ones_chunk, zeros_deg)
    dis, g1 = _tc1(degp, x, W1)
    s1p = _edge_kernel(g1, src, dst, zeros_feat)
    g2 = _tc2(s1p, g1, dis, b1.reshape(1, D), W2)
    s2p = _edge_kernel(g2, src, dst, zeros_feat)
    out = _tc3(s2p, g2, dis, b2.reshape(1, D))
    return out


# speedup vs baseline: 33.3040x; 1.1991x over previous
import functools

import jax
import jax.numpy as jnp
from jax import lax
from jax.experimental import pallas as pl
from jax.experimental.pallas import tpu as pltpu
from jax.experimental.pallas import tpu_sc as plsc

N = 10000
D = 128
DH = D // 2
E = 320000

NC = 2
NS = 16
NW = NC * NS
EPT = E // NW
K = 128
NCH = 80
EPTP = NCH * K
PAD = EPTP - EPT
NBUF = 8
LOOK = 6
LAG = NBUF - LOOK
NP = 10240
RPT = NP // NS
DW = 16

_MESH = dict(core_axis_name="c", subcore_axis_name="s", num_cores=NC,
             num_subcores=NS)


@functools.partial(
    pl.kernel,
    out_type=jax.ShapeDtypeStruct((NC, NP, DW), jnp.float32),
    mesh=plsc.VectorSubcoreMesh(**_MESH),
    compiler_params=pltpu.CompilerParams(use_tc_tiling_on_sc=False),
    scratch_types=[
        pltpu.VMEM((NCH, K), jnp.int32),
        pltpu.VMEM((K, DW), jnp.float32),
        pltpu.VMEM_SHARED((NP, DW), jnp.float32),
    ],
)
def _deg_kernel(dst_hbm, ones_hbm, zeros_hbm, out_hbm, idx_d, ones_v, acc):
    c = lax.axis_index("c")
    s = lax.axis_index("s")
    wid = c * NS + s

    pltpu.sync_copy(dst_hbm.at[wid], idx_d)
    pltpu.sync_copy(ones_hbm, ones_v)
    pltpu.sync_copy(zeros_hbm.at[pl.ds(s * RPT, RPT)],
                    acc.at[pl.ds(s * RPT, RPT)])
    plsc.subcore_barrier()

    def body(j, _):
        pltpu.sync_copy(ones_v, acc.at[idx_d.at[j]], add=True)
        return _

    lax.fori_loop(0, NCH, body, None)
    plsc.subcore_barrier()

    pltpu.sync_copy(acc.at[pl.ds(s * RPT, RPT)],
                    out_hbm.at[c, pl.ds(s * RPT, RPT)])


@functools.partial(
    pl.kernel,
    out_type=jax.ShapeDtypeStruct((NC, NP, D), jnp.float32),
    mesh=plsc.VectorSubcoreMesh(**_MESH),
    compiler_params=pltpu.CompilerParams(use_tc_tiling_on_sc=False),
    scratch_types=[
        pltpu.VMEM((NCH, K), jnp.int32),
        pltpu.VMEM((NCH, K), jnp.int32),
        [pltpu.VMEM((K, DH), jnp.float32) for _ in range(NBUF)],
        [pltpu.SemaphoreType.DMA for _ in range(NBUF)],
        [pltpu.SemaphoreType.DMA for _ in range(NBUF)],
        pltpu.VMEM_SHARED((NP, DH), jnp.float32),
    ],
)
def _edge_kernel(g2n_hbm, srcl_hbm, dst_hbm, zeros_hbm, out_hbm,
                 idx_s, idx_d, rows, gsem, ssem, acc):
    c = lax.axis_index("c")
    s = lax.axis_index("s")
    wid = c * NS + s

    pltpu.sync_copy(srcl_hbm.at[wid], idx_s)
    pltpu.sync_copy(dst_hbm.at[wid], idx_d)

    for h in range(2):
        if h == 1:
            def bump(r, _):
                for cc in range(K // 16):
                    sl = pl.ds(cc * 16, 16)
                    idx_s[r, sl] = idx_s[r, sl] + 1
                return _

            lax.fori_loop(0, NCH, bump, None)

        pltpu.sync_copy(zeros_hbm.at[pl.ds(s * RPT, RPT)],
                        acc.at[pl.ds(s * RPT, RPT)])
        plsc.subcore_barrier()

        for b in range(LOOK):
            pltpu.async_copy(g2n_hbm.at[idx_s.at[b]], rows[b], gsem[b])

        def body(j, _):
            for bi in range(NBUF):
                ch = NBUF * j + bi
                pltpu.make_async_copy(g2n_hbm.at[idx_s.at[ch]],
                                      rows[bi], gsem[bi]).wait()
                pltpu.async_copy(rows[bi], acc.at[idx_d.at[ch]], ssem[bi],
                                 add=True)
                bw = (bi - LAG) % NBUF

                @pl.when(ch >= LAG)
                def _():
                    pltpu.make_async_copy(rows[bw], acc.at[idx_d.at[ch]],
                                          ssem[bw]).wait()

                bg = (bi + LOOK) % NBUF

                @pl.when(ch + LOOK < NCH)
                def _():
                    pltpu.async_copy(g2n_hbm.at[idx_s.at[ch + LOOK]],
                                     rows[bg], gsem[bg])
            return _

        lax.fori_loop(0, NCH // NBUF, body, None)
        for ch in range(NCH - LAG, NCH):
            bi = ch % NBUF
            pltpu.make_async_copy(rows[bi], acc.at[idx_d.at[NCH - 1]],
                                  ssem[bi]).wait()
        plsc.subcore_barrier()

        pltpu.sync_copy(acc.at[pl.ds(s * RPT, RPT)],
                        out_hbm.at[c, pl.ds(s * RPT, RPT), pl.ds(h * DH, DH)])


NB = 10
BR = N // NB


def _tc1_body(degp_ref, x_ref, w1_ref, dis_ref, g1_ref):
    deg = degp_ref[0, :, :1] + degp_ref[1, :, :1] + 1.0
    dis = lax.rsqrt(deg)
    h1 = jnp.dot(x_ref[...], w1_ref[...],
                 preferred_element_type=jnp.float32,
                 precision=lax.Precision.HIGHEST)
    dis_ref[...] = dis
    g1_ref[...] = dis * h1


def _tc2_body(s1p_ref, g1_ref, dis_ref, b1_ref, w2_ref, g2_ref):
    dis = dis_ref[...]
    t = (s1p_ref[0] + s1p_ref[1] + g1_ref[...]) * dis + b1_ref[...]
    z = jnp.maximum(t, 0.0)
    h2 = jnp.dot(z, w2_ref[...],
                 preferred_element_type=jnp.float32,
                 precision=lax.Precision.HIGHEST)
    g2_ref[...] = dis * h2


def _tc3_body(s2p_ref, g2_ref, dis_ref, b2_ref, out_ref):
    out_ref[...] = ((s2p_ref[0] + s2p_ref[1] + g2_ref[...]) * dis_ref[...]
                    + b2_ref[...])


_spec_degp = pl.BlockSpec((NC, BR, DW), lambda i: (0, i, 0))
_spec_sp = pl.BlockSpec((NC, BR, D), lambda i: (0, i, 0))
_spec_dis = pl.BlockSpec((BR, 1), lambda i: (i, 0))
_spec_row = pl.BlockSpec((BR, D), lambda i: (i, 0))
_spec_b = pl.BlockSpec((1, D), lambda i: (0, 0))
_spec_w = pl.BlockSpec((D, D), lambda i: (0, 0))

_tc1 = pl.pallas_call(
    _tc1_body,
    grid=(NB,),
    in_specs=[_spec_degp, _spec_row, _spec_w],
    out_specs=(_spec_dis, _spec_row),
    out_shape=(jax.ShapeDtypeStruct((N, 1), jnp.float32),
               jax.ShapeDtypeStruct((N, D), jnp.float32)),
)

_tc2 = pl.pallas_call(
    _tc2_body,
    grid=(NB,),
    in_specs=[_spec_sp, _spec_row, _spec_dis, _spec_b, _spec_w],
    out_specs=_spec_row,
    out_shape=jax.ShapeDtypeStruct((N, D), jnp.float32),
)

_tc3 = pl.pallas_call(
    _tc3_body,
    grid=(NB,),
    in_specs=[_spec_sp, _spec_row, _spec_dis, _spec_b],
    out_specs=_spec_row,
    out_shape=jax.ShapeDtypeStruct((N, D), jnp.float32),
)


def kernel(x, edge_index, batch, W1, b1, W2, b2):
    del batch
    src = edge_index[0].reshape(NW, EPT)
    dst = edge_index[1].reshape(NW, EPT)
    pad_src = jnp.broadcast_to(2 * (jnp.arange(PAD, dtype=jnp.int32) % N),
                               (NW, PAD))
    pad_dst = jnp.broadcast_to(N + (jnp.arange(PAD, dtype=jnp.int32) % (NP - N)),
                               (NW, PAD))
    srcl = jnp.concatenate([2 * src, pad_src], axis=1).reshape(NW, NCH, K)
    dstp = jnp.concatenate([dst, pad_dst], axis=1).reshape(NW, NCH, K)

    zeros_feat = jnp.zeros((NP, DH), jnp.float32)
    zeros_deg = jnp.zeros((NP, DW), jnp.float32)
    ones_chunk = jnp.ones((K, DW), jnp.float32)

    degp = _deg_kernel(dstp, ones_chunk, zeros_deg)
    dis, g1 = _tc1(degp, x, W1)
    s1 = _edge_kernel(g1.reshape(2 * N, DH), srcl, dstp, zeros_feat)
    g2 = _tc2(s1, g1, dis, b1.reshape(1, D), W2)
    s2 = _edge_kernel(g2.reshape(2 * N, DH), srcl, dstp, zeros_feat)
    out = _tc3(s2, g2, dis, b2.reshape(1, D))
    return out


# speedup vs baseline: 36.5483x; 1.0974x over previous
import functools

import jax
import jax.numpy as jnp
from jax import lax
from jax.experimental import pallas as pl
from jax.experimental.pallas import tpu as pltpu
from jax.experimental.pallas import tpu_sc as plsc

N = 10000
D = 128
E = 320000

NC = 2
NS = 16
NW = NC * NS
EPT = E // NW
K = 128
NCH = 80
EPTP = NCH * K
PAD = EPTP - EPT
NBUF = 8
LOOK = 6
LAG = NBUF - LOOK
NP = 10240
RPT = NP // NS
DW = 16

_MESH = dict(core_axis_name="c", subcore_axis_name="s", num_cores=NC,
             num_subcores=NS)


@functools.partial(
    pl.kernel,
    out_type=jax.ShapeDtypeStruct((NC, NP, DW), jnp.float32),
    mesh=plsc.VectorSubcoreMesh(**_MESH),
    compiler_params=pltpu.CompilerParams(use_tc_tiling_on_sc=False),
    scratch_types=[
        pltpu.VMEM((NCH, K), jnp.int32),
        pltpu.VMEM((K, DW), jnp.float32),
        pltpu.VMEM_SHARED((NP, DW), jnp.float32),
    ],
)
def _deg_kernel(dst_hbm, ones_hbm, zeros_hbm, out_hbm, idx_d, ones_v, acc):
    c = lax.axis_index("c")
    s = lax.axis_index("s")
    wid = c * NS + s

    pltpu.sync_copy(dst_hbm.at[wid], idx_d)
    pltpu.sync_copy(ones_hbm, ones_v)
    pltpu.sync_copy(zeros_hbm.at[pl.ds(s * RPT, RPT)],
                    acc.at[pl.ds(s * RPT, RPT)])
    plsc.subcore_barrier()

    def body(j, _):
        pltpu.sync_copy(ones_v, acc.at[idx_d.at[j]], add=True)
        return _

    lax.fori_loop(0, NCH, body, None)
    plsc.subcore_barrier()

    pltpu.sync_copy(acc.at[pl.ds(s * RPT, RPT)],
                    out_hbm.at[c, pl.ds(s * RPT, RPT)])


@functools.partial(
    pl.kernel,
    out_type=jax.ShapeDtypeStruct((NC, NP, D), jnp.bfloat16),
    mesh=plsc.VectorSubcoreMesh(**_MESH),
    compiler_params=pltpu.CompilerParams(use_tc_tiling_on_sc=False),
    scratch_types=[
        pltpu.VMEM((NCH, K), jnp.int32),
        pltpu.VMEM((NCH, K), jnp.int32),
        [pltpu.VMEM((K, D), jnp.bfloat16) for _ in range(NBUF)],
        [pltpu.SemaphoreType.DMA for _ in range(NBUF)],
        [pltpu.SemaphoreType.DMA for _ in range(NBUF)],
        pltpu.VMEM_SHARED((NP, D), jnp.bfloat16),
    ],
)
def _edge_kernel(g_hbm, src_hbm, dst_hbm, zeros_hbm, out_hbm,
                 idx_s, idx_d, rows, gsem, ssem, acc):
    c = lax.axis_index("c")
    s = lax.axis_index("s")
    wid = c * NS + s

    pltpu.sync_copy(src_hbm.at[wid], idx_s)
    pltpu.sync_copy(dst_hbm.at[wid], idx_d)
    pltpu.sync_copy(zeros_hbm.at[pl.ds(s * RPT, RPT)],
                    acc.at[pl.ds(s * RPT, RPT)])
    plsc.subcore_barrier()

    for b in range(LOOK):
        pltpu.async_copy(g_hbm.at[idx_s.at[b]], rows[b], gsem[b])

    def body(j, _):
        for bi in range(NBUF):
            ch = NBUF * j + bi
            pltpu.make_async_copy(g_hbm.at[idx_s.at[ch]],
                                  rows[bi], gsem[bi]).wait()
            pltpu.async_copy(rows[bi], acc.at[idx_d.at[ch]], ssem[bi],
                             add=True)
            bw = (bi - LAG) % NBUF

            @pl.when(ch >= LAG)
            def _():
                pltpu.make_async_copy(rows[bw], acc.at[idx_d.at[ch]],
                                      ssem[bw]).wait()

            bg = (bi + LOOK) % NBUF

            @pl.when(ch + LOOK < NCH)
            def _():
                pltpu.async_copy(g_hbm.at[idx_s.at[ch + LOOK]],
                                 rows[bg], gsem[bg])
        return _

    lax.fori_loop(0, NCH // NBUF, body, None)
    for ch in range(NCH - LAG, NCH):
        bi = ch % NBUF
        pltpu.make_async_copy(rows[bi], acc.at[idx_d.at[NCH - 1]],
                              ssem[bi]).wait()
    plsc.subcore_barrier()

    pltpu.sync_copy(acc.at[pl.ds(s * RPT, RPT)],
                    out_hbm.at[c, pl.ds(s * RPT, RPT)])


NB = 10
BR = N // NB


def _tc1_body(degp_ref, x_ref, w1_ref, dis_ref, g1_ref, gb1_ref):
    deg = degp_ref[0, :, :1] + degp_ref[1, :, :1] + 1.0
    dis = lax.rsqrt(deg)
    h1 = jnp.dot(x_ref[...], w1_ref[...],
                 preferred_element_type=jnp.float32,
                 precision=lax.Precision.HIGHEST)
    g1 = dis * h1
    dis_ref[...] = dis
    g1_ref[...] = g1
    gb1_ref[...] = g1.astype(jnp.bfloat16)


def _tc2_body(s1p_ref, g1_ref, dis_ref, b1_ref, w2_ref, g2_ref, gb2_ref):
    dis = dis_ref[...]
    sp = s1p_ref[0].astype(jnp.float32) + s1p_ref[1].astype(jnp.float32)
    t = (sp + g1_ref[...]) * dis + b1_ref[...]
    z = jnp.maximum(t, 0.0)
    h2 = jnp.dot(z, w2_ref[...],
                 preferred_element_type=jnp.float32,
                 precision=lax.Precision.HIGHEST)
    g2 = dis * h2
    g2_ref[...] = g2
    gb2_ref[...] = g2.astype(jnp.bfloat16)


def _tc3_body(s2p_ref, g2_ref, dis_ref, b2_ref, out_ref):
    sp = s2p_ref[0].astype(jnp.float32) + s2p_ref[1].astype(jnp.float32)
    out_ref[...] = (sp + g2_ref[...]) * dis_ref[...] + b2_ref[...]


_spec_degp = pl.BlockSpec((NC, BR, DW), lambda i: (0, i, 0))
_spec_sp = pl.BlockSpec((NC, BR, D), lambda i: (0, i, 0))
_spec_dis = pl.BlockSpec((BR, 1), lambda i: (i, 0))
_spec_row = pl.BlockSpec((BR, D), lambda i: (i, 0))
_spec_b = pl.BlockSpec((1, D), lambda i: (0, 0))
_spec_w = pl.BlockSpec((D, D), lambda i: (0, 0))

_tc1 = pl.pallas_call(
    _tc1_body,
    grid=(NB,),
    in_specs=[_spec_degp, _spec_row, _spec_w],
    out_specs=(_spec_dis, _spec_row, _spec_row),
    out_shape=(jax.ShapeDtypeStruct((N, 1), jnp.float32),
               jax.ShapeDtypeStruct((N, D), jnp.float32),
               jax.ShapeDtypeStruct((N, D), jnp.bfloat16)),
)

_tc2 = pl.pallas_call(
    _tc2_body,
    grid=(NB,),
    in_specs=[_spec_sp, _spec_row, _spec_dis, _spec_b, _spec_w],
    out_specs=(_spec_row, _spec_row),
    out_shape=(jax.ShapeDtypeStruct((N, D), jnp.float32),
               jax.ShapeDtypeStruct((N, D), jnp.bfloat16)),
)

_tc3 = pl.pallas_call(
    _tc3_body,
    grid=(NB,),
    in_specs=[_spec_sp, _spec_row, _spec_dis, _spec_b],
    out_specs=_spec_row,
    out_shape=jax.ShapeDtypeStruct((N, D), jnp.float32),
)


def kernel(x, edge_index, batch, W1, b1, W2, b2):
    del batch
    src = edge_index[0].reshape(NW, EPT)
    dst = edge_index[1].reshape(NW, EPT)
    pad_src = jnp.broadcast_to(jnp.arange(PAD, dtype=jnp.int32) % N,
                               (NW, PAD))
    pad_dst = jnp.broadcast_to(N + (jnp.arange(PAD, dtype=jnp.int32) % (NP - N)),
                               (NW, PAD))
    srcp = jnp.concatenate([src, pad_src], axis=1).reshape(NW, NCH, K)
    dstp = jnp.concatenate([dst, pad_dst], axis=1).reshape(NW, NCH, K)

    zeros_feat = jnp.zeros((NP, D), jnp.bfloat16)
    zeros_deg = jnp.zeros((NP, DW), jnp.float32)
    ones_chunk = jnp.ones((K, DW), jnp.float32)

    degp = _deg_kernel(dstp, ones_chunk, zeros_deg)
    dis, g1, gb1 = _tc1(degp, x, W1)
    s1 = _edge_kernel(gb1, srcp, dstp, zeros_feat)
    g2, gb2 = _tc2(s1, g1, dis, b1.reshape(1, D), W2)
    s2 = _edge_kernel(gb2, srcp, dstp, zeros_feat)
    out = _tc3(s2, g2, dis, b2.reshape(1, D))
    return out
